# baseline (speedup 1.0000x reference)
import functools

import jax
import jax.numpy as jnp
import numpy as np
from jax.experimental import pallas as pl
from jax.experimental.pallas import tpu as pltpu

EPS_BN = 1e-5


def _batch_gather(x, idx):
    B, C, N = x.shape
    _, P, K = idx.shape
    idxe = jnp.broadcast_to(idx[:, None, :, :], (B, C, P, K)).reshape(B, C, P * K)
    return jnp.take_along_axis(x, idxe, axis=2).reshape(B, C, P, K)


def _bn(p, x):
    return (x - p['rm'][None, :, None]) / jnp.sqrt(p['rv'][None, :, None] + EPS_BN) * p['g'][None, :, None] + p['b'][None, :, None]


def _conv1d(p, x):
    return jnp.einsum('oc,bcn->bon', p['w'], x) + p['b'][None, :, None]


def _fka_conv(p, x, pos, support, idx):
    posg = _batch_gather(pos, idx)
    xg = _batch_gather(x, idx)
    pts = posg - support[:, :, :, None]
    distances = jnp.sqrt(jnp.sum(pts ** 2, axis=1))
    pts = pts / p['norm_radius']
    dw = jax.nn.sigmoid(-p['alpha'] * distances + p['beta'])
    dws = jnp.sum(dw, axis=2, keepdims=True)
    dws = dws + (dws == 0).astype(dw.dtype) + 1e-6
    dw = (dw / dws * distances.shape[2])[:, None, :, :]
    mat = jax.nn.relu(jnp.einsum('oc,bcpk->bopk', p['fc1'], pts))
    mp1 = jnp.broadcast_to(jnp.max(mat * dw, axis=3, keepdims=True), mat.shape)
    mat = jnp.concatenate([mat, mp1], axis=1)
    mat = jax.nn.relu(jnp.einsum('oc,bcpk->bopk', p['fc2'], mat))
    mp2 = jnp.broadcast_to(jnp.max(mat * dw, axis=3, keepdims=True), mat.shape)
    mat = jnp.concatenate([mat, mp2], axis=1)
    mat = jax.nn.relu(jnp.einsum('oc,bcpk->bopk', p['fc3'], mat))
    mat = mat * dw
    aligned = jnp.einsum('bcpk,bspk->bcps', xg, mat)
    out = jnp.einsum('ocs,bcps->bop', p['cv'], aligned)
    if 'cv_b' in p:
        out = out + p['cv_b'][None, :, None]
    return out


def _res_block(p, x, pos, support, idx):
    x_short = x
    h = jax.nn.relu(_bn(p['bn0'], _conv1d(p['cv0'], x)))
    h = jax.nn.relu(_bn(p['bn1'], _fka_conv(p['cv1'], h, pos, support, idx)))
    h = _bn(p['bn2'], _conv1d(p['cv2'], h))
    if x_short.shape[2] != h.shape[2]:
        x_short = jnp.max(_batch_gather(x_short, idx), axis=3)
    if 'shortcut' in p:
        x_short = _conv1d(p['shortcut'], x_short)
    return jax.nn.relu(h + x_short)


def _head_kernel(x4_ref, w_ref, b_ref, out_ref):
    x4 = x4_ref[...]
    xm = jnp.mean(x4, axis=2)
    out_ref[...] = jnp.dot(xm, w_ref[...].T, preferred_element_type=jnp.float32) + b_ref[...][None, :]


def _head(x4, w, b):
    B = x4.shape[0]
    O = w.shape[0]
    return pl.pallas_call(
        _head_kernel,
        out_shape=jax.ShapeDtypeStruct((B, O), jnp.float32),
    )(x4, w, b)


def kernel(x, pos, support1, support2, support3, support4, ids0, ids10, ids11, ids20, ids21, ids30, ids31, ids40, ids41, params):
    s1, s2, s3, s4 = support1, support2, support3, support4
    x0 = jax.nn.relu(_bn(params['bn0'], _fka_conv(params['cv0'], x, pos, pos, ids0)))
    x0 = _res_block(params['b01'], x0, pos, pos, ids0)
    x1 = _res_block(params['b10'], x0, pos, s1, ids10)
    x1 = _res_block(params['b11'], x1, s1, s1, ids11)
    x2 = _res_block(params['b20'], x1, s1, s2, ids20)
    x2 = _res_block(params['b21'], x2, s2, s2, ids21)
    x3 = _res_block(params['b30'], x2, s2, s3, ids30)
    x3 = _res_block(params['b31'], x3, s3, s3, ids31)
    x4 = _res_block(params['b40'], x3, s3, s4, ids40)
    x4 = _res_block(params['b41'], x4, s4, s4, ids41)
    return _head(x4, params['fcout']['w'], params['fcout']['b'])


# speedup vs baseline: 324.8406x; 324.8406x over previous
import functools

import jax
import jax.numpy as jnp
from jax import lax
from jax.experimental import pallas as pl
from jax.experimental.pallas import tpu as pltpu
from jax.experimental.pallas import tpu_sc as plsc

EPS_BN = 1e-5
_NC, _NS = 2, 16
_NW = _NC * _NS


@functools.lru_cache(maxsize=None)
def _sc_gather(rows_table, n_out, d):
    per_sub = n_out // _NW
    assert n_out % _NW == 0
    r = 128
    while r > 8 and r * d * 4 * 2 > 760_000 // 2:
        r //= 2
    r = min(r, per_sub)
    iters = per_sub // r
    assert per_sub % r == 0, (per_sub, r)
    mesh = plsc.VectorSubcoreMesh(core_axis_name="c", subcore_axis_name="s")

    @functools.partial(
        pl.kernel,
        mesh=mesh,
        compiler_params=pltpu.CompilerParams(use_tc_tiling_on_sc=False),
        out_type=jax.ShapeDtypeStruct((n_out, d), jnp.float32),
        scratch_types=[
            pltpu.VMEM((r,), jnp.int32),
            pltpu.VMEM((r, d), jnp.float32),
            pltpu.SemaphoreType.DMA,
        ],
    )
    def gather_k(table_hbm, idx_hbm, out_hbm, idx_v, rows_v, sem):
        wid = lax.axis_index("s") * _NC + lax.axis_index("c")
        base = wid * per_sub

        def body(i, carry):
            off = base + i * r
            pltpu.sync_copy(idx_hbm.at[pl.ds(off, r)], idx_v)
            pltpu.async_copy(table_hbm.at[idx_v], rows_v, sem).wait()
            pltpu.sync_copy(rows_v, out_hbm.at[pl.ds(off, r)])
            return carry

        lax.fori_loop(0, iters, body, 0)

    return gather_k


def _gather_rows(table, idx):
    B, N, D = table.shape
    _, P, K = idx.shape
    idxf = (idx.astype(jnp.int32) + (jnp.arange(B, dtype=jnp.int32) * N)[:, None, None])
    g = _sc_gather(B * N, B * P * K, D)(table.reshape(B * N, D), idxf.reshape(-1))
    return g.reshape(B, P, K, D)


def _pad16(c):
    return (c + 15) // 16 * 16


def _build_table(parts):
    t = jnp.concatenate(parts, axis=-1)
    d = _pad16(t.shape[-1])
    if d != t.shape[-1]:
        t = jnp.pad(t, ((0, 0), (0, 0), (0, d - t.shape[-1])))
    return t


def _bn_rows(p, x):
    scale = p['g'] / jnp.sqrt(p['rv'] + EPS_BN)
    return (x - p['rm']) * scale + p['b']


def _conv_rows(p, x):
    return x @ p['w'].T + p['b']


def _fka_math(p, g, c_feat, pos_off, sup_rows):
    B, P, K, _ = g.shape
    xg = g[..., :c_feat]
    posg = g[..., pos_off:pos_off + 3]
    pts = posg - sup_rows[:, :, None, :]
    dist = jnp.sqrt(jnp.sum(pts * pts, axis=-1))
    pts = pts / p['norm_radius']
    dw = jax.nn.sigmoid(-p['alpha'] * dist + p['beta'])
    dws = jnp.sum(dw, axis=-1, keepdims=True)
    dws = dws + (dws == 0).astype(dw.dtype) + 1e-6
    dw = dw / dws * K
    dwe = dw[..., None]
    mat = jax.nn.relu(jnp.einsum('bpkc,sc->bpks', pts, p['fc1']))
    mp1 = jnp.max(mat * dwe, axis=2, keepdims=True)
    mat = jax.nn.relu(jnp.einsum('bpkc,sc->bpks', mat, p['fc2'][:, :16])
                      + jnp.einsum('bpkc,sc->bpks', mp1, p['fc2'][:, 16:]))
    mp2 = jnp.max(mat * dwe, axis=2, keepdims=True)
    mat = jax.nn.relu(jnp.einsum('bpkc,sc->bpks', mat, p['fc3'][:, :16])
                      + jnp.einsum('bpkc,sc->bpks', mp2, p['fc3'][:, 16:]))
    mat = mat * dwe
    aligned = jnp.einsum('bpkc,bpks->bpcs', xg, mat)
    O = p['cv'].shape[0]
    out = aligned.reshape(B, P, c_feat * 16) @ p['cv'].reshape(O, c_feat * 16).T
    if 'cv_b' in p:
        out = out + p['cv_b']
    return out


def _res_block(p, xr, pos_rows, sup_rows, idx):
    B, N, cin = xr.shape
    P = idx.shape[1]
    down = (N != P)
    h = jax.nn.relu(_bn_rows(p['bn0'], _conv_rows(p['cv0'], xr)))
    c2 = h.shape[-1]
    parts = [h, xr, pos_rows] if down else [h, pos_rows]
    g = _gather_rows(_build_table(parts), idx)
    pos_off = c2 + (cin if down else 0)
    f = _fka_math(p['cv1'], g, c2, pos_off, sup_rows)
    f = jax.nn.relu(_bn_rows(p['bn1'], f))
    f = _bn_rows(p['bn2'], _conv_rows(p['cv2'], f))
    xs = jnp.max(g[..., c2:c2 + cin], axis=2) if down else xr
    if 'shortcut' in p:
        xs = _conv_rows(p['shortcut'], xs)
    return jax.nn.relu(f + xs)


def _head_kernel(x4_ref, w_ref, b_ref, out_ref):
    xm = jnp.mean(x4_ref[...], axis=1)
    out_ref[...] = jnp.dot(xm, w_ref[...].T, preferred_element_type=jnp.float32) + b_ref[...][None, :]


def kernel(x, pos, support1, support2, support3, support4, ids0, ids10, ids11, ids20, ids21, ids30, ids31, ids40, ids41, params):
    xr = x.transpose(0, 2, 1)
    pr = pos.transpose(0, 2, 1)
    s1r = support1.transpose(0, 2, 1)
    s2r = support2.transpose(0, 2, 1)
    s3r = support3.transpose(0, 2, 1)
    s4r = support4.transpose(0, 2, 1)

    g0 = _gather_rows(_build_table([xr, pr]), ids0)
    x0 = _fka_math(params['cv0'], g0, 3, 3, pr)
    x0 = jax.nn.relu(_bn_rows(params['bn0'], x0))
    x0 = _res_block(params['b01'], x0, pr, pr, ids0)
    x1 = _res_block(params['b10'], x0, pr, s1r, ids10)
    x1 = _res_block(params['b11'], x1, s1r, s1r, ids11)
    x2 = _res_block(params['b20'], x1, s1r, s2r, ids20)
    x2 = _res_block(params['b21'], x2, s2r, s2r, ids21)
    x3 = _res_block(params['b30'], x2, s2r, s3r, ids30)
    x3 = _res_block(params['b31'], x3, s3r, s3r, ids31)
    x4 = _res_block(params['b40'], x3, s3r, s4r, ids40)
    x4 = _res_block(params['b41'], x4, s4r, s4r, ids41)

    w = params['fcout']['w']
    return pl.pallas_call(
        _head_kernel,
        out_shape=jax.ShapeDtypeStruct((x4.shape[0], w.shape[0]), jnp.float32),
    )(x4, w, params['fcout']['b'])
